# baseline (speedup 1.0000x reference)
import functools

import jax
import jax.numpy as jnp
from jax import lax
from jax.experimental import pallas as pl
from jax.experimental.pallas import tpu as pltpu
from jax.experimental.pallas import tpu_sc as plsc

B = 4096
L = 200
D = 32
NC = 2
NS = 16
NW = NC * NS
LT = L // 8
HROWS = 4 * 128


def _sc_embed_body(x4_hbm, table_hbm, out5_hbm, idx_v, rows_v, out_v,
                   gsem0, gsem1, wsem0, wsem1):
    w = lax.axis_index("s") * NC + lax.axis_index("c")

    def idx_cp(lt):
        return pltpu.make_async_copy(x4_hbm.at[lt, w], idx_v.at[lt], wsem0)

    def idx_fire(lt, carry):
        idx_cp(lt).start()
        return carry

    def idx_drain(lt, carry):
        idx_cp(lt).wait()
        return carry

    lax.fori_loop(0, LT, idx_fire, 0)
    lax.fori_loop(0, LT, idx_drain, 0)

    lane = lax.iota(jnp.int32, 16)
    lane_hi = lane + 16
    gsems = (gsem0, gsem1)
    wsems = (wsem0, wsem1)

    def gather_cps(lt, h):
        return [
            pltpu.make_async_copy(
                table_hbm.at[idx_v.at[lt, 4 * h + k]],
                rows_v.at[h].at[pl.ds(k * 128, 128)],
                gsems[h],
            )
            for k in range(4)
        ]

    def write_cp(lt, j):
        return pltpu.make_async_copy(
            out_v.at[j], out5_hbm.at[:, lt, w], wsems[j]
        )

    def transpose_half(h, j):
        base = jnp.int32(512 * h)

        def tbody(r, c):
            v0 = rows_v[h, r, pl.ds(0, 16)]
            v1 = rows_v[h, r, pl.ds(16, 16)]
            rf = jnp.full((16,), base + r, jnp.int32)
            plsc.store_scatter(out_v.at[j], [lane, rf], v0)
            plsc.store_scatter(out_v.at[j], [lane_hi, rf], v1)
            return c

        lax.fori_loop(0, HROWS, tbody, 0, unroll=8)

    for cp in gather_cps(0, 0) + gather_cps(0, 1):
        cp.start()

    def lt_body(lt, carry):
        for j in (0, 1):

            @pl.when((lt >= 2) & (lt % 2 == j))
            def _():
                write_cp(lt - 2, j).wait()

        for h in (0, 1):
            for cp in gather_cps(lt, h):
                cp.wait()

            for j in (0, 1):

                @pl.when(lt % 2 == j)
                def _():
                    transpose_half(h, j)

            @pl.when(lt + 1 < LT)
            def _():
                for cp in gather_cps(lt + 1, h):
                    cp.start()

        for j in (0, 1):

            @pl.when(lt % 2 == j)
            def _():
                write_cp(lt, j).start()

        return carry

    lax.fori_loop(0, LT, lt_body, 0)
    write_cp(LT - 2, LT % 2).wait()
    write_cp(LT - 1, (LT - 1) % 2).wait()


@jax.jit
def _embed(x4, table):
    mesh = plsc.VectorSubcoreMesh(
        core_axis_name="c", subcore_axis_name="s", num_cores=NC, num_subcores=NS
    )
    return pl.kernel(
        _sc_embed_body,
        out_type=jax.ShapeDtypeStruct((D, LT, NW, 1024), jnp.float32),
        mesh=mesh,
        scratch_types=[
            pltpu.VMEM((LT, 8, 128), jnp.int32),
            pltpu.VMEM((2, HROWS, D), jnp.float32),
            pltpu.VMEM((2, D, 1024), jnp.float32),
            pltpu.SemaphoreType.DMA,
            pltpu.SemaphoreType.DMA,
            pltpu.SemaphoreType.DMA,
            pltpu.SemaphoreType.DMA,
        ],
        compiler_params=pltpu.CompilerParams(
            needs_layout_passes=False, use_tc_tiling_on_sc=False
        ),
    )(x4, table)


def kernel(x, table):
    x4 = x.astype(jnp.int32).T.reshape(LT, 8, NW, 128).transpose(0, 2, 1, 3)
    out5 = _embed(x4, table)
    return (
        out5.reshape(D, LT, NW, 8, 128)
        .transpose(2, 4, 0, 1, 3)
        .reshape(B, D, L)
    )


# speedup vs baseline: 1.5999x; 1.5999x over previous
import functools

import jax
import jax.numpy as jnp
from jax import lax
from jax.experimental import pallas as pl
from jax.experimental.pallas import tpu as pltpu
from jax.experimental.pallas import tpu_sc as plsc

B = 4096
L = 200
D = 32
NC = 2
NS = 16
NW = NC * NS
LT = L // 8
HROWS = 4 * 128


def _sc_embed_body(x4_hbm, table_hbm, out5_hbm, idx_v, rows_v, out_v,
                   gsem0, gsem1, wsem0, wsem1):
    w = lax.axis_index("s") * NC + lax.axis_index("c")

    def idx_cp(lt):
        return pltpu.make_async_copy(x4_hbm.at[lt, w], idx_v.at[lt], wsem0)

    def idx_fire(lt, carry):
        idx_cp(lt).start()
        return carry

    def idx_drain(lt, carry):
        idx_cp(lt).wait()
        return carry

    lax.fori_loop(0, LT, idx_fire, 0)
    lax.fori_loop(0, LT, idx_drain, 0)

    lane = lax.iota(jnp.int32, 16)
    lane_hi = lane + 16
    gsems = (gsem0, gsem1)
    wsems = (wsem0, wsem1)

    def gather_cps(lt, h):
        return [
            pltpu.make_async_copy(
                table_hbm.at[idx_v.at[lt, 4 * h + k]],
                rows_v.at[h].at[pl.ds(k * 128, 128)],
                gsems[h],
            )
            for k in range(4)
        ]

    def write_cp(lt, j):
        return pltpu.make_async_copy(
            out_v.at[j, :, pl.ds(0, 1024)], out5_hbm.at[:, lt, w], wsems[j]
        )

    def transpose_half(h, j):
        def tbody(r, rfv):
            v0 = rows_v[h, r, pl.ds(0, 16)]
            v1 = rows_v[h, r, pl.ds(16, 16)]
            plsc.store_scatter(out_v.at[j], [lane, rfv], v0)
            plsc.store_scatter(out_v.at[j], [lane_hi, rfv], v1)
            return rfv + 1

        lax.fori_loop(
            0, HROWS, tbody, jnp.full((16,), 512 * h, jnp.int32), unroll=8
        )

    for cp in gather_cps(0, 0) + gather_cps(0, 1):
        cp.start()

    def lt_body(lt, carry):
        for j in (0, 1):

            @pl.when((lt >= 2) & (lt % 2 == j))
            def _():
                write_cp(lt - 2, j).wait()

        for h in (0, 1):
            for cp in gather_cps(lt, h):
                cp.wait()

            for j in (0, 1):

                @pl.when(lt % 2 == j)
                def _():
                    transpose_half(h, j)

            @pl.when(lt + 1 < LT)
            def _():
                for cp in gather_cps(lt + 1, h):
                    cp.start()

        for j in (0, 1):

            @pl.when(lt % 2 == j)
            def _():
                write_cp(lt, j).start()

        return carry

    lax.fori_loop(0, LT, lt_body, 0)
    write_cp(LT - 2, LT % 2).wait()
    write_cp(LT - 1, (LT - 1) % 2).wait()


@jax.jit
def _embed(x4, table):
    mesh = plsc.VectorSubcoreMesh(
        core_axis_name="c", subcore_axis_name="s", num_cores=NC, num_subcores=NS
    )
    return pl.kernel(
        _sc_embed_body,
        out_type=jax.ShapeDtypeStruct((D, LT, NW, 1024), jnp.float32),
        mesh=mesh,
        scratch_types=[
            pltpu.VMEM((LT, 8, 128), jnp.int32),
            pltpu.VMEM((2, HROWS, D), jnp.float32),
            pltpu.VMEM((2, D, 1025), jnp.float32),
            pltpu.SemaphoreType.DMA,
            pltpu.SemaphoreType.DMA,
            pltpu.SemaphoreType.DMA,
            pltpu.SemaphoreType.DMA,
        ],
        compiler_params=pltpu.CompilerParams(
            needs_layout_passes=False, use_tc_tiling_on_sc=False
        ),
    )(x4, table)


def kernel(x, table):
    x4 = x.astype(jnp.int32).T.reshape(LT, 8, NW, 128).transpose(0, 2, 1, 3)
    out5 = _embed(x4, table)
    return (
        out5.reshape(D, LT, NW, 8, 128)
        .transpose(2, 4, 0, 1, 3)
        .reshape(B, D, L)
    )


# speedup vs baseline: 1.6946x; 1.0592x over previous
import functools

import jax
import jax.numpy as jnp
from jax import lax
from jax.experimental import pallas as pl
from jax.experimental.pallas import tpu as pltpu
from jax.experimental.pallas import tpu_sc as plsc

B = 4096
L = 200
D = 32
NC = 2
NS = 16
NW = NC * NS
LT = L // 8
HROWS = 4 * 128


def _sc_embed_body(x4_hbm, table_hbm, out5_hbm, idx_v, rows_v, out_v,
                   gsem0, gsem1, wsem0, wsem1):
    w = lax.axis_index("s") * NC + lax.axis_index("c")

    def idx_cp(lt):
        return pltpu.make_async_copy(x4_hbm.at[lt, w], idx_v.at[lt], wsem0)

    def idx_fire(lt, carry):
        idx_cp(lt).start()
        return carry

    def idx_drain(lt, carry):
        idx_cp(lt).wait()
        return carry

    lax.fori_loop(0, LT, idx_fire, 0)
    lax.fori_loop(0, LT, idx_drain, 0)

    lane = lax.iota(jnp.int32, 16)
    lane_hi = lane + 16
    gsems = (gsem0, gsem1)
    wsems = (wsem0, wsem1)

    def gather_cps(lt, h):
        return [
            pltpu.make_async_copy(
                table_hbm.at[idx_v.at[lt, 4 * h + k]],
                rows_v.at[h].at[pl.ds(k * 128, 128)],
                gsems[h],
            )
            for k in range(4)
        ]

    def write_cp(lt, j):
        return pltpu.make_async_copy(
            out_v.at[j, :, pl.ds(0, 1024)], out5_hbm.at[:, lt, w], wsems[j]
        )

    def transpose_half(h, j):
        def tbody(r4, rfv):
            r0 = r4 * 4
            vs = [
                (rows_v[h, r0 + k, pl.ds(0, 16)],
                 rows_v[h, r0 + k, pl.ds(16, 16)])
                for k in range(4)
            ]
            for k, (v0, v1) in enumerate(vs):
                rf = rfv + k
                plsc.store_scatter(out_v.at[j], [lane, rf], v0)
                plsc.store_scatter(out_v.at[j], [lane_hi, rf], v1)
            return rfv + 4

        lax.fori_loop(
            0, HROWS // 4, tbody, jnp.full((16,), 512 * h, jnp.int32),
            unroll=4,
        )

    for cp in gather_cps(0, 0) + gather_cps(0, 1):
        cp.start()

    def lt_body(lt, carry):
        for j in (0, 1):

            @pl.when((lt >= 2) & (lt % 2 == j))
            def _():
                write_cp(lt - 2, j).wait()

        for h in (0, 1):
            for cp in gather_cps(lt, h):
                cp.wait()

            for j in (0, 1):

                @pl.when(lt % 2 == j)
                def _():
                    transpose_half(h, j)

            @pl.when(lt + 1 < LT)
            def _():
                for cp in gather_cps(lt + 1, h):
                    cp.start()

        for j in (0, 1):

            @pl.when(lt % 2 == j)
            def _():
                write_cp(lt, j).start()

        return carry

    lax.fori_loop(0, LT, lt_body, 0)
    write_cp(LT - 2, LT % 2).wait()
    write_cp(LT - 1, (LT - 1) % 2).wait()


@jax.jit
def _embed(x4, table):
    mesh = plsc.VectorSubcoreMesh(
        core_axis_name="c", subcore_axis_name="s", num_cores=NC, num_subcores=NS
    )
    return pl.kernel(
        _sc_embed_body,
        out_type=jax.ShapeDtypeStruct((D, LT, NW, 1024), jnp.float32),
        mesh=mesh,
        scratch_types=[
            pltpu.VMEM((LT, 8, 128), jnp.int32),
            pltpu.VMEM((2, HROWS, D), jnp.float32),
            pltpu.VMEM((2, D, 1025), jnp.float32),
            pltpu.SemaphoreType.DMA,
            pltpu.SemaphoreType.DMA,
            pltpu.SemaphoreType.DMA,
            pltpu.SemaphoreType.DMA,
        ],
        compiler_params=pltpu.CompilerParams(
            needs_layout_passes=False, use_tc_tiling_on_sc=False
        ),
    )(x4, table)


def kernel(x, table):
    x4 = x.astype(jnp.int32).T.reshape(LT, 8, NW, 128).transpose(0, 2, 1, 3)
    out5 = _embed(x4, table)
    return (
        out5.reshape(D, LT, NW, 8, 128)
        .transpose(2, 4, 0, 1, 3)
        .reshape(B, D, L)
    )
